# baseline (speedup 1.0000x reference)
import functools

import jax
import jax.numpy as jnp
from jax import lax
from jax.experimental import pallas as pl
from jax.experimental.pallas import tpu as pltpu
from jax.experimental.pallas import tpu_sc as plsc

BATCH = 16384
EMB = 64
NC = 2
NS = 16
L = 16
NW = NC * NS
BPW = BATCH // NW
NG = BPW // L


def _body(user_hbm, ad_hbm, utab_hbm, atab_hbm, fcw_hbm, fcb_hbm, out_hbm,
          uidx_v, aidx_v, urows_v, arows_v, fcw_v, fcb_v, outbuf_v, sem):
    wid = lax.axis_index("s") * NC + lax.axis_index("c")
    base = wid * BPW

    pltpu.sync_copy(user_hbm.at[pl.ds(base, BPW)], uidx_v)
    pltpu.sync_copy(ad_hbm.at[pl.ds(base, BPW)], aidx_v)
    pltpu.sync_copy(fcw_hbm, fcw_v)
    pltpu.sync_copy(fcb_hbm, fcb_v)

    def fetch(g, carry):
        uidx = uidx_v[pl.ds(g * L, L)] * EMB
        aidx = aidx_v[pl.ds(g * L, L)] * EMB
        handles = []
        for l in range(L):
            k = g * L + l
            uoff = pl.multiple_of(uidx[l], EMB)
            aoff = pl.multiple_of(aidx[l], EMB)
            handles.append(pltpu.make_async_copy(
                utab_hbm.at[pl.ds(uoff, EMB)],
                urows_v.at[pl.ds(k * EMB, EMB)], sem))
            handles.append(pltpu.make_async_copy(
                atab_hbm.at[pl.ds(aoff, EMB)],
                arows_v.at[pl.ds(k * EMB, EMB)], sem))
        for h in handles:
            h.start()
        for h in handles:
            h.wait()
        return carry

    lax.fori_loop(0, NG, fetch, 0)

    iot = lax.iota(jnp.int32, L)
    wv = fcw_v[...]
    bv = fcb_v[...]

    def group(g, carry):
        row_off = (g * L + iot) * EMB
        acc_ua = jnp.zeros((L,), jnp.float32)
        acc_uu = jnp.zeros((L,), jnp.float32)
        acc_aa = jnp.zeros((L,), jnp.float32)
        for d in range(EMB):
            idx = row_off + d
            u = plsc.load_gather(urows_v, [idx])
            a = plsc.load_gather(arows_v, [idx])
            acc_ua = acc_ua + u * a
            acc_uu = acc_uu + u * u
            acc_aa = acc_aa + a * a
        x = jnp.maximum(acc_uu * acc_aa, jnp.float32(1e-30))
        i = lax.bitcast_convert_type(x, jnp.int32)
        i = jnp.int32(0x5F3759DF) - lax.shift_right_logical(i, 1)
        y = lax.bitcast_convert_type(i, jnp.float32)
        for _ in range(3):
            y = y * (jnp.float32(1.5) - jnp.float32(0.5) * x * y * y)
        dot = acc_ua * y
        z = dot * wv + bv
        s = jnp.float32(1.0) / (jnp.float32(1.0) + jnp.exp(-z))
        outbuf_v[pl.ds(g * L, L)] = s
        return carry

    lax.fori_loop(0, NG, group, 0)

    pltpu.sync_copy(outbuf_v, out_hbm.at[pl.ds(base, BPW)])


@jax.jit
def _run(user, ad, utab_flat, atab_flat, wvec, bvec):
    mesh = plsc.VectorSubcoreMesh(core_axis_name="c", subcore_axis_name="s")
    k = functools.partial(
        pl.kernel,
        mesh=mesh,
        compiler_params=pltpu.CompilerParams(
            use_tc_tiling_on_sc=False, needs_layout_passes=False),
        out_type=jax.ShapeDtypeStruct((BATCH,), jnp.float32),
        scratch_types=[
            pltpu.VMEM((BPW,), jnp.int32),
            pltpu.VMEM((BPW,), jnp.int32),
            pltpu.VMEM((BPW * EMB,), jnp.float32),
            pltpu.VMEM((BPW * EMB,), jnp.float32),
            pltpu.VMEM((L,), jnp.float32),
            pltpu.VMEM((L,), jnp.float32),
            pltpu.VMEM((BPW,), jnp.float32),
            pltpu.SemaphoreType.DMA,
        ],
    )(_body)
    return k(user, ad, utab_flat, atab_flat, wvec, bvec)


def kernel(user, ad, user_table, ad_table, fc_w, fc_b):
    user = user.astype(jnp.int32)
    ad = ad.astype(jnp.int32)
    utab_flat = user_table.reshape(-1)
    atab_flat = ad_table.reshape(-1)
    wvec = jnp.broadcast_to(fc_w.reshape(()), (L,)).astype(jnp.float32)
    bvec = jnp.broadcast_to(fc_b.reshape(()), (L,)).astype(jnp.float32)
    out = _run(user, ad, utab_flat, atab_flat, wvec, bvec)
    return out.reshape(BATCH, 1)


# speedup vs baseline: 1.6041x; 1.6041x over previous
import functools

import jax
import jax.numpy as jnp
from jax import lax
from jax.experimental import pallas as pl
from jax.experimental.pallas import tpu as pltpu
from jax.experimental.pallas import tpu_sc as plsc

BATCH = 16384
EMB = 64
NC = 2
NS = 16
L = 16
NW = NC * NS
BPW = BATCH // NW
CHR = 256
NPASS = BPW // CHR
NGC = CHR // L


def _body(user_hbm, ad_hbm, utab_hbm, atab_hbm, fcw_hbm, fcb_hbm, out_hbm,
          uidx_v, aidx_v, urows_v, arows_v, fcw_v, fcb_v, outbuf_v, sem):
    wid = lax.axis_index("s") * NC + lax.axis_index("c")
    base = wid * BPW

    pltpu.sync_copy(user_hbm.at[pl.ds(base, BPW)], uidx_v)
    pltpu.sync_copy(ad_hbm.at[pl.ds(base, BPW)], aidx_v)
    pltpu.sync_copy(fcw_hbm, fcw_v)
    pltpu.sync_copy(fcb_hbm, fcb_v)

    iot = lax.iota(jnp.int32, L)
    wv = fcw_v[...]
    bv = fcb_v[...]

    for p in range(NPASS):
        def make_group(g, p=p):
            uidx = uidx_v[pl.ds((p * NGC + g) * L, L)]
            aidx = aidx_v[pl.ds((p * NGC + g) * L, L)]
            handles = []
            for l in range(L):
                k = g * L + l
                handles.append(pltpu.make_async_copy(
                    utab_hbm.at[uidx[l]], urows_v.at[k], sem))
                handles.append(pltpu.make_async_copy(
                    atab_hbm.at[aidx[l]], arows_v.at[k], sem))
            return handles

        def fetch(g, carry, p=p):
            for h in make_group(g, p):
                h.start()
            @pl.when(g > 0)
            def _():
                for h in make_group(g - 1, p):
                    h.wait()
            return carry

        lax.fori_loop(0, NGC, fetch, 0)
        for h in make_group(NGC - 1, p):
            h.wait()

        def group(g, carry, p=p):
            row16 = g * L + iot
            acc_ua = jnp.zeros((L,), jnp.float32)
            acc_uu = jnp.zeros((L,), jnp.float32)
            acc_aa = jnp.zeros((L,), jnp.float32)
            for d in range(EMB):
                dsp = jnp.full((L,), d, jnp.int32)
                u = plsc.load_gather(urows_v, [row16, dsp])
                a = plsc.load_gather(arows_v, [row16, dsp])
                acc_ua = acc_ua + u * a
                acc_uu = acc_uu + u * u
                acc_aa = acc_aa + a * a
            x = jnp.maximum(acc_uu * acc_aa, jnp.float32(1e-30))
            i = lax.bitcast_convert_type(x, jnp.int32)
            i = jnp.int32(0x5F3759DF) - lax.shift_right_logical(i, 1)
            y = lax.bitcast_convert_type(i, jnp.float32)
            for _ in range(3):
                y = y * (jnp.float32(1.5) - jnp.float32(0.5) * x * y * y)
            dot = acc_ua * y
            z = dot * wv + bv
            s = jnp.float32(1.0) / (jnp.float32(1.0) + jnp.exp(-z))
            outbuf_v[pl.ds((p * NGC + g) * L, L)] = s
            return carry

        lax.fori_loop(0, NGC, group, 0)

    pltpu.sync_copy(outbuf_v, out_hbm.at[pl.ds(base, BPW)])


@jax.jit
def _run(user, ad, user_table, ad_table, wvec, bvec):
    mesh = plsc.VectorSubcoreMesh(core_axis_name="c", subcore_axis_name="s")
    k = functools.partial(
        pl.kernel,
        mesh=mesh,
        compiler_params=pltpu.CompilerParams(needs_layout_passes=False),
        out_type=jax.ShapeDtypeStruct((BATCH,), jnp.float32),
        scratch_types=[
            pltpu.VMEM((BPW,), jnp.int32),
            pltpu.VMEM((BPW,), jnp.int32),
            pltpu.VMEM((CHR, EMB), jnp.float32),
            pltpu.VMEM((CHR, EMB), jnp.float32),
            pltpu.VMEM((L,), jnp.float32),
            pltpu.VMEM((L,), jnp.float32),
            pltpu.VMEM((BPW,), jnp.float32),
            pltpu.SemaphoreType.DMA,
        ],
    )(_body)
    return k(user, ad, user_table, ad_table, wvec, bvec)


def kernel(user, ad, user_table, ad_table, fc_w, fc_b):
    user = user.astype(jnp.int32)
    ad = ad.astype(jnp.int32)
    wvec = jnp.broadcast_to(fc_w.reshape(()), (L,)).astype(jnp.float32)
    bvec = jnp.broadcast_to(fc_b.reshape(()), (L,)).astype(jnp.float32)
    out = _run(user, ad, user_table, ad_table, wvec, bvec)
    return out.reshape(BATCH, 1)


# speedup vs baseline: 3.2826x; 2.0464x over previous
import functools

import jax
import jax.numpy as jnp
from jax import lax
from jax.experimental import pallas as pl
from jax.experimental.pallas import tpu as pltpu
from jax.experimental.pallas import tpu_sc as plsc

BATCH = 16384
EMB = 64
NC = 2
NS = 16
L = 16
NW = NC * NS
BPW = BATCH // NW
NROW_U = 1000000
NROW_A = 100000
PANEL = 128
TAIL_U = NROW_U // PANEL
TAILW_U = NROW_U - TAIL_U * PANEL
TAIL_A = NROW_A // PANEL
TAILW_A = NROW_A - TAIL_A * PANEL
RING_P = 4
RING_R = 16


def _stage_body(ru_hbm, pu_hbm, ra_hbm, pa_hbm, utab_hbm, atab_hbm,
                ustage_hbm, astage_hbm,
                rbuf, pbbuf, dlist, pbuf, rowring, sem_p, sem_r):
    wid = lax.axis_index("s") * NC + lax.axis_index("c")
    base = wid * BPW
    iot = lax.iota(jnp.int32, L)

    def one_table(r_hbm, p_hbm, tab_hbm, stage_hbm):
        pltpu.sync_copy(r_hbm.at[pl.ds(base, BPW)], rbuf.at[pl.ds(0, BPW)])
        pltpu.sync_copy(p_hbm.at[pl.ds(base, BPW)], pbbuf.at[pl.ds(0, BPW)])

        def scan_g(g, carry):
            cur, dc, acc = carry
            pv = lax.shift_right_logical(rbuf[pl.ds(g * L, L)], 7)
            for l in range(L):
                p = pv[l]
                new = p != cur
                acc = jnp.where((iot == lax.rem(dc, L)) & new, p, acc)

                @pl.when(new & (lax.rem(dc, L) == L - 1))
                def _(acc=acc, dc=dc):
                    dlist[pl.ds(dc - (L - 1), L)] = acc

                dc = dc + new.astype(jnp.int32)
                cur = p
            return cur, dc, acc

        cur, dcnt, acc = lax.fori_loop(
            0, BPW // L, scan_g,
            (jnp.int32(-1), jnp.int32(0), jnp.zeros((L,), jnp.int32)))
        dlist[pl.ds((dcnt // L) * L, L)] = acc

        def panel_at(d):
            return dlist[pl.ds(d, L)][0]

        def full_handle(s, p):
            col = pl.multiple_of(p * PANEL, PANEL)
            return pltpu.make_async_copy(
                tab_hbm.at[:, pl.ds(col, PANEL)], pbuf.at[s], sem_p)

        def start_panel(d, s):
            @pl.when(d < dcnt)
            def _():
                full_handle(s, panel_at(d)).start()

        for s in range(RING_P - 1):
            start_panel(jnp.int32(s), s)

        def row_wait_handle(q):
            return pltpu.make_async_copy(
                rowring.at[pl.ds(q * EMB, EMB)],
                stage_hbm.at[pl.ds(0, EMB)], sem_r)

        def proc_block(dblk, carry):
            eptr, srow = carry
            for s in range(RING_P):
                d = dblk * RING_P + s

                def do_panel(eptr, srow, d=d, s=s):
                    p = panel_at(d)
                    full_handle(s, p).wait()
                    start_panel(d + RING_P - 1, (s + RING_P - 1) % RING_P)

                    def w_cond(c):
                        e, sr = c
                        r = rbuf[pl.ds(e, L)][0]
                        return (e < BPW) & (
                            lax.shift_right_logical(r, 7) == p)

                    def w_body(c):
                        e, sr = c
                        r = rbuf[pl.ds(e, L)][0]
                        b = pbbuf[pl.ds(e, L)][0]
                        rloc = lax.rem(r, PANEL)
                        q = lax.rem(sr, RING_R)
                        ssp = jnp.full((L,), s, jnp.int32)
                        rsp = jnp.zeros((L,), jnp.int32) + rloc
                        for kk in range(EMB // L):
                            g = plsc.load_gather(
                                pbuf, [ssp, iot + kk * L, rsp])
                            rowring[pl.ds(q * EMB + kk * L, L)] = g
                        pltpu.make_async_copy(
                            rowring.at[pl.ds(q * EMB, EMB)],
                            stage_hbm.at[pl.ds(
                                pl.multiple_of(b * EMB, 8), EMB)],
                            sem_r).start()

                        @pl.when(sr >= RING_R)
                        def _():
                            row_wait_handle(q).wait()

                        return e + 1, sr + 1

                    return lax.while_loop(w_cond, w_body, (eptr, srow))

                guard = jnp.where(d < dcnt, 1, 0)
                eptr, srow = lax.cond(
                    guard > 0, lambda c: do_panel(c[0], c[1]),
                    lambda c: c, (eptr, srow))
            return eptr, srow

        nblk = (dcnt + RING_P - 1) // RING_P
        eptr, srow = lax.fori_loop(
            0, nblk, proc_block, (jnp.int32(0), jnp.int32(0)))

        def drain(i, carry):
            row_wait_handle(lax.rem(i, RING_R)).wait()
            return carry

        lax.fori_loop(0, jnp.minimum(srow, RING_R), drain, 0)

    one_table(ru_hbm, pu_hbm, utab_hbm, ustage_hbm)
    one_table(ra_hbm, pa_hbm, atab_hbm, astage_hbm)


def _compute_body(ustage_hbm, astage_hbm, fcw_hbm, fcb_hbm, out_hbm,
                  urows_v, arows_v, fcw_v, fcb_v, outbuf_v):
    wid = lax.axis_index("s") * NC + lax.axis_index("c")
    base = wid * BPW

    pltpu.sync_copy(ustage_hbm.at[pl.ds(base * EMB, BPW * EMB)], urows_v)
    pltpu.sync_copy(astage_hbm.at[pl.ds(base * EMB, BPW * EMB)], arows_v)
    pltpu.sync_copy(fcw_hbm, fcw_v)
    pltpu.sync_copy(fcb_hbm, fcb_v)

    iot = lax.iota(jnp.int32, L)
    wv = fcw_v[...]
    bv = fcb_v[...]

    def group(g, carry):
        row_off = (g * L + iot) * EMB
        acc_ua = jnp.zeros((L,), jnp.float32)
        acc_uu = jnp.zeros((L,), jnp.float32)
        acc_aa = jnp.zeros((L,), jnp.float32)
        for d in range(EMB):
            idx = row_off + d
            u = plsc.load_gather(urows_v, [idx])
            a = plsc.load_gather(arows_v, [idx])
            acc_ua = acc_ua + u * a
            acc_uu = acc_uu + u * u
            acc_aa = acc_aa + a * a
        x = jnp.maximum(acc_uu * acc_aa, jnp.float32(1e-30))
        i = lax.bitcast_convert_type(x, jnp.int32)
        i = jnp.int32(0x5F3759DF) - lax.shift_right_logical(i, 1)
        y = lax.bitcast_convert_type(i, jnp.float32)
        for _ in range(3):
            y = y * (jnp.float32(1.5) - jnp.float32(0.5) * x * y * y)
        dot = acc_ua * y
        z = dot * wv + bv
        s = jnp.float32(1.0) / (jnp.float32(1.0) + jnp.exp(-z))
        outbuf_v[pl.ds(g * L, L)] = s
        return carry

    lax.fori_loop(0, BPW // L, group, 0)

    pltpu.sync_copy(outbuf_v, out_hbm.at[pl.ds(base, BPW)])


@jax.jit
def _run(ru, pu, ra, pa, utab_t, atab_t, wvec, bvec):
    mesh = plsc.VectorSubcoreMesh(core_axis_name="c", subcore_axis_name="s")
    stage = functools.partial(
        pl.kernel,
        mesh=mesh,
        compiler_params=pltpu.CompilerParams(needs_layout_passes=False),
        out_type=(jax.ShapeDtypeStruct((BATCH * EMB,), jnp.float32),
                  jax.ShapeDtypeStruct((BATCH * EMB,), jnp.float32)),
        scratch_types=[
            pltpu.VMEM((BPW + 2 * L,), jnp.int32),
            pltpu.VMEM((BPW + 2 * L,), jnp.int32),
            pltpu.VMEM((BPW + 2 * L,), jnp.int32),
            pltpu.VMEM((RING_P, EMB, PANEL), jnp.float32),
            pltpu.VMEM((RING_R * EMB,), jnp.float32),
            pltpu.SemaphoreType.DMA,
            pltpu.SemaphoreType.DMA,
        ],
    )(_stage_body)
    ustage, astage = stage(ru, pu, ra, pa, utab_t, atab_t)

    comp = functools.partial(
        pl.kernel,
        mesh=mesh,
        compiler_params=pltpu.CompilerParams(needs_layout_passes=False),
        out_type=jax.ShapeDtypeStruct((BATCH,), jnp.float32),
        scratch_types=[
            pltpu.VMEM((BPW * EMB,), jnp.float32),
            pltpu.VMEM((BPW * EMB,), jnp.float32),
            pltpu.VMEM((L,), jnp.float32),
            pltpu.VMEM((L,), jnp.float32),
            pltpu.VMEM((BPW,), jnp.float32),
        ],
    )(_compute_body)
    return comp(ustage, astage, wvec, bvec)


def kernel(user, ad, user_table, ad_table, fc_w, fc_b):
    user = user.astype(jnp.int32)
    ad = ad.astype(jnp.int32)
    iota_b = jnp.arange(BATCH, dtype=jnp.int32)
    ru, pu = lax.sort_key_val(user, iota_b)
    ra, pa = lax.sort_key_val(ad, iota_b)
    utab_t = jnp.transpose(user_table)
    atab_t = jnp.transpose(ad_table)
    wvec = jnp.broadcast_to(fc_w.reshape(()), (L,)).astype(jnp.float32)
    bvec = jnp.broadcast_to(fc_b.reshape(()), (L,)).astype(jnp.float32)
    out = _run(ru, pu, ra, pa, utab_t, atab_t, wvec, bvec)
    return out.reshape(BATCH, 1)


# speedup vs baseline: 3.5334x; 1.0764x over previous
import functools

import jax
import jax.numpy as jnp
from jax import lax
from jax.experimental import pallas as pl
from jax.experimental.pallas import tpu as pltpu
from jax.experimental.pallas import tpu_sc as plsc

BATCH = 16384
EMB = 64
NC = 2
NS = 16
L = 16
NW = NC * NS
BPW = BATCH // NW
NROW_U = 1000000
NROW_A = 100000
PANEL = 128
TAIL_U = NROW_U // PANEL
TAILW_U = NROW_U - TAIL_U * PANEL
TAIL_A = NROW_A // PANEL
TAILW_A = NROW_A - TAIL_A * PANEL
RING_P = 8
RING_R = 16


def _stage_body(ru_hbm, pu_hbm, ra_hbm, pa_hbm, utab_hbm, atab_hbm,
                ustage_hbm, astage_hbm,
                rbuf, pbbuf, dlist, pbuf, rowring, sem_p, sem_r):
    wid = lax.axis_index("s") * NC + lax.axis_index("c")
    base = wid * BPW
    iot = lax.iota(jnp.int32, L)

    def one_table(r_hbm, p_hbm, tab_hbm, stage_hbm):
        pltpu.sync_copy(r_hbm.at[pl.ds(base, BPW)], rbuf.at[pl.ds(0, BPW)])
        pltpu.sync_copy(p_hbm.at[pl.ds(base, BPW)], pbbuf.at[pl.ds(0, BPW)])

        def scan_g(g, carry):
            cur, dc, acc = carry
            pv = lax.shift_right_logical(rbuf[pl.ds(g * L, L)], 7)
            for l in range(L):
                p = pv[l]
                new = p != cur
                acc = jnp.where((iot == lax.rem(dc, L)) & new, p, acc)

                @pl.when(new & (lax.rem(dc, L) == L - 1))
                def _(acc=acc, dc=dc):
                    dlist[pl.ds(dc - (L - 1), L)] = acc

                dc = dc + new.astype(jnp.int32)
                cur = p
            return cur, dc, acc

        cur, dcnt, acc = lax.fori_loop(
            0, BPW // L, scan_g,
            (jnp.int32(-1), jnp.int32(0), jnp.zeros((L,), jnp.int32)))
        dlist[pl.ds((dcnt // L) * L, L)] = acc

        def panel_at(d):
            return dlist[pl.ds(d, L)][0]

        def full_handle(s, p):
            col = pl.multiple_of(p * PANEL, PANEL)
            return pltpu.make_async_copy(
                tab_hbm.at[:, pl.ds(col, PANEL)], pbuf.at[s], sem_p)

        def start_panel(d, s):
            @pl.when(d < dcnt)
            def _():
                full_handle(s, panel_at(d)).start()

        for s in range(RING_P - 1):
            start_panel(jnp.int32(s), s)

        def row_wait_handle(q):
            return pltpu.make_async_copy(
                rowring.at[pl.ds(q * EMB, EMB)],
                stage_hbm.at[pl.ds(0, EMB)], sem_r)

        def proc_block(dblk, carry):
            eptr, srow = carry
            for s in range(RING_P):
                d = dblk * RING_P + s

                def do_panel(eptr, srow, d=d, s=s):
                    p = panel_at(d)
                    full_handle(s, p).wait()
                    start_panel(d + RING_P - 1, (s + RING_P - 1) % RING_P)

                    def w_cond(c):
                        e, sr = c
                        r = rbuf[pl.ds(e, L)][0]
                        return (e < BPW) & (
                            lax.shift_right_logical(r, 7) == p)

                    def w_body(c):
                        e, sr = c
                        r = rbuf[pl.ds(e, L)][0]
                        b = pbbuf[pl.ds(e, L)][0]
                        rloc = lax.rem(r, PANEL)
                        q = lax.rem(sr, RING_R)
                        ssp = jnp.full((L,), s, jnp.int32)
                        rsp = jnp.zeros((L,), jnp.int32) + rloc
                        for kk in range(EMB // L):
                            g = plsc.load_gather(
                                pbuf, [ssp, iot + kk * L, rsp])
                            rowring[pl.ds(q * EMB + kk * L, L)] = g
                        pltpu.make_async_copy(
                            rowring.at[pl.ds(q * EMB, EMB)],
                            stage_hbm.at[pl.ds(
                                pl.multiple_of(b * EMB, 8), EMB)],
                            sem_r).start()

                        @pl.when(sr >= RING_R)
                        def _():
                            row_wait_handle(q).wait()

                        return e + 1, sr + 1

                    return lax.while_loop(w_cond, w_body, (eptr, srow))

                guard = jnp.where(d < dcnt, 1, 0)
                eptr, srow = lax.cond(
                    guard > 0, lambda c: do_panel(c[0], c[1]),
                    lambda c: c, (eptr, srow))
            return eptr, srow

        nblk = (dcnt + RING_P - 1) // RING_P
        eptr, srow = lax.fori_loop(
            0, nblk, proc_block, (jnp.int32(0), jnp.int32(0)))

        def drain(i, carry):
            row_wait_handle(lax.rem(i, RING_R)).wait()
            return carry

        lax.fori_loop(0, jnp.minimum(srow, RING_R), drain, 0)

    one_table(ru_hbm, pu_hbm, utab_hbm, ustage_hbm)
    one_table(ra_hbm, pa_hbm, atab_hbm, astage_hbm)


def _compute_body(ustage_hbm, astage_hbm, fcw_hbm, fcb_hbm, out_hbm,
                  urows_v, arows_v, fcw_v, fcb_v, outbuf_v):
    wid = lax.axis_index("s") * NC + lax.axis_index("c")
    base = wid * BPW

    pltpu.sync_copy(ustage_hbm.at[pl.ds(base * EMB, BPW * EMB)], urows_v)
    pltpu.sync_copy(astage_hbm.at[pl.ds(base * EMB, BPW * EMB)], arows_v)
    pltpu.sync_copy(fcw_hbm, fcw_v)
    pltpu.sync_copy(fcb_hbm, fcb_v)

    iot = lax.iota(jnp.int32, L)
    wv = fcw_v[...]
    bv = fcb_v[...]

    def group(g, carry):
        row_off = (g * L + iot) * EMB
        acc_ua = jnp.zeros((L,), jnp.float32)
        acc_uu = jnp.zeros((L,), jnp.float32)
        acc_aa = jnp.zeros((L,), jnp.float32)
        for d in range(EMB):
            idx = row_off + d
            u = plsc.load_gather(urows_v, [idx])
            a = plsc.load_gather(arows_v, [idx])
            acc_ua = acc_ua + u * a
            acc_uu = acc_uu + u * u
            acc_aa = acc_aa + a * a
        x = jnp.maximum(acc_uu * acc_aa, jnp.float32(1e-30))
        i = lax.bitcast_convert_type(x, jnp.int32)
        i = jnp.int32(0x5F3759DF) - lax.shift_right_logical(i, 1)
        y = lax.bitcast_convert_type(i, jnp.float32)
        for _ in range(3):
            y = y * (jnp.float32(1.5) - jnp.float32(0.5) * x * y * y)
        dot = acc_ua * y
        z = dot * wv + bv
        s = jnp.float32(1.0) / (jnp.float32(1.0) + jnp.exp(-z))
        outbuf_v[pl.ds(g * L, L)] = s
        return carry

    lax.fori_loop(0, BPW // L, group, 0)

    pltpu.sync_copy(outbuf_v, out_hbm.at[pl.ds(base, BPW)])


@jax.jit
def _run(ru, pu, ra, pa, utab_t, atab_t, wvec, bvec):
    mesh = plsc.VectorSubcoreMesh(core_axis_name="c", subcore_axis_name="s")
    stage = functools.partial(
        pl.kernel,
        mesh=mesh,
        compiler_params=pltpu.CompilerParams(needs_layout_passes=False),
        out_type=(jax.ShapeDtypeStruct((BATCH * EMB,), jnp.float32),
                  jax.ShapeDtypeStruct((BATCH * EMB,), jnp.float32)),
        scratch_types=[
            pltpu.VMEM((BPW + 2 * L,), jnp.int32),
            pltpu.VMEM((BPW + 2 * L,), jnp.int32),
            pltpu.VMEM((BPW + 2 * L,), jnp.int32),
            pltpu.VMEM((RING_P, EMB, PANEL), jnp.float32),
            pltpu.VMEM((RING_R * EMB,), jnp.float32),
            pltpu.SemaphoreType.DMA,
            pltpu.SemaphoreType.DMA,
        ],
    )(_stage_body)
    ustage, astage = stage(ru, pu, ra, pa, utab_t, atab_t)

    comp = functools.partial(
        pl.kernel,
        mesh=mesh,
        compiler_params=pltpu.CompilerParams(needs_layout_passes=False),
        out_type=jax.ShapeDtypeStruct((BATCH,), jnp.float32),
        scratch_types=[
            pltpu.VMEM((BPW * EMB,), jnp.float32),
            pltpu.VMEM((BPW * EMB,), jnp.float32),
            pltpu.VMEM((L,), jnp.float32),
            pltpu.VMEM((L,), jnp.float32),
            pltpu.VMEM((BPW,), jnp.float32),
        ],
    )(_compute_body)
    return comp(ustage, astage, wvec, bvec)


def kernel(user, ad, user_table, ad_table, fc_w, fc_b):
    user = user.astype(jnp.int32)
    ad = ad.astype(jnp.int32)
    iota_b = jnp.arange(BATCH, dtype=jnp.int32)
    ru, pu = lax.sort_key_val(user, iota_b)
    ra, pa = lax.sort_key_val(ad, iota_b)
    utab_t = jnp.transpose(user_table)
    atab_t = jnp.transpose(ad_table)
    wvec = jnp.broadcast_to(fc_w.reshape(()), (L,)).astype(jnp.float32)
    bvec = jnp.broadcast_to(fc_b.reshape(()), (L,)).astype(jnp.float32)
    out = _run(ru, pu, ra, pa, utab_t, atab_t, wvec, bvec)
    return out.reshape(BATCH, 1)
